# baseline (speedup 1.0000x reference)
import jax
import jax.numpy as jnp
from jax.experimental import pallas as pl
from jax.experimental.pallas import tpu as pltpu

_PLACEHOLDER_TOKEN = 42
_B, _N, _D = 1024, 77, 768
_CH = 16
_NC = _B // _CH
_NB = 4


def _ring_kernel(tokT_ref, emb_hbm, ph_ref, out_hbm,
                 inbuf, outbuf, in_sems, out_sems):
    ph = ph_ref[...]

    def start_in(i, slot):
        pltpu.make_async_copy(
            emb_hbm.at[pl.ds(i * _CH, _CH)], inbuf.at[slot], in_sems.at[slot]
        ).start()

    def wait_in(i, slot):
        pltpu.make_async_copy(
            emb_hbm.at[pl.ds(i * _CH, _CH)], inbuf.at[slot], in_sems.at[slot]
        ).wait()

    def start_out(i, slot):
        pltpu.make_async_copy(
            outbuf.at[slot], out_hbm.at[pl.ds(i * _CH, _CH)], out_sems.at[slot]
        ).start()

    def wait_out(i, slot):
        pltpu.make_async_copy(
            outbuf.at[slot], out_hbm.at[pl.ds(i * _CH, _CH)], out_sems.at[slot]
        ).wait()

    for s in range(_NB):
        start_in(s, s)

    for i in range(_NC):
        slot = i % _NB
        wait_in(i, slot)
        if i >= _NB:
            wait_out(i - _NB, slot)
        maskT = tokT_ref[i] == _PLACEHOLDER_TOKEN
        for b in range(_CH):
            outbuf[slot, b] = jnp.where(maskT[:, b : b + 1], ph, inbuf[slot, b])
        start_out(i, slot)
        nxt = i + _NB
        if nxt < _NC:
            start_in(nxt, slot)

    for i in range(_NC - _NB, _NC):
        wait_out(i, i % _NB)


def kernel(tokenized_text, embedded_text, placeholder_embedding):
    tokT = tokenized_text.reshape(_NC, _CH, _N).transpose(0, 2, 1)
    return pl.pallas_call(
        _ring_kernel,
        in_specs=[
            pl.BlockSpec(memory_space=pltpu.VMEM),
            pl.BlockSpec(memory_space=pltpu.MemorySpace.HBM),
            pl.BlockSpec(memory_space=pltpu.VMEM),
        ],
        out_specs=pl.BlockSpec(memory_space=pltpu.MemorySpace.HBM),
        out_shape=jax.ShapeDtypeStruct((_B, _N, _D), embedded_text.dtype),
        scratch_shapes=[
            pltpu.VMEM((_NB, _CH, _N, _D), jnp.float32),
            pltpu.VMEM((_NB, _CH, _N, _D), jnp.float32),
            pltpu.SemaphoreType.DMA((_NB,)),
            pltpu.SemaphoreType.DMA((_NB,)),
        ],
    )(tokT, embedded_text, placeholder_embedding)
